# Initial kernel scaffold; baseline (speedup 1.0000x reference)
import jax
import jax.numpy as jnp
from jax.experimental import pallas as pl


def kernel(x, pos, batch, params):
    raise NotImplementedError("write your pallas kernel here")


# speedup vs baseline: 2.0567x; 2.0567x over previous
import math
from functools import partial

import jax
import jax.numpy as jnp
from jax.experimental import pallas as pl
from jax.experimental.pallas import tpu as pltpu

_INP_DIM = 3
_OUP_DIM = 13
_MAXNB = 128

_SA_SPECS = [
    (0.2, 0.05, [_INP_DIM + 3, 64, 64, 128]),
    (0.3, 0.1, [131, 128, 128, 128]),
    (0.3, 0.2, [131, 256, 256, 256]),
    (0.3, 0.4, [259, 512, 512, 512]),
    (0.3, 0.8, [515, 512, 512, 1024]),
]
_FP_SPECS = [
    (1, [1536, 512, 512]),
    (3, [768, 256, 256]),
    (3, [384, 256, 256]),
    (3, [384, 256, 128]),
    (3, [128 + _INP_DIM, 128, 128, 128]),
]


def _pad_to(n, m):
    return ((n + m - 1) // m) * m


def _fps_kernel(pxyz_ref, pr_ref, o_ref, *, n_s):
    X = pxyz_ref[0:1, :]
    Y = pxyz_ref[1:2, :]
    Z = pxyz_ref[2:3, :]
    Np = X.shape[1]
    iota = jax.lax.broadcasted_iota(jnp.int32, (1, Np), 1)
    r0 = pr_ref[0:1, :]
    px = r0[0:1, 0:1]
    py = r0[0:1, 1:2]
    pz = r0[0:1, 2:3]
    d = (X - px) ** 2 + (Y - py) ** 2 + (Z - pz) ** 2
    ns_pad = o_ref.shape[1]
    jota = jax.lax.broadcasted_iota(jnp.int32, (1, ns_pad), 1)
    idxv = jnp.zeros((1, ns_pad), jnp.int32)

    def body(i, carry):
        d, idxv = carry
        m = jnp.max(d)
        nxt = jnp.min(jnp.where(d == m, iota, jnp.int32(Np)))
        row = pr_ref[pl.ds(nxt, 1), :]
        qx = row[0:1, 0:1]
        qy = row[0:1, 1:2]
        qz = row[0:1, 2:3]
        dn = (X - qx) ** 2 + (Y - qy) ** 2 + (Z - qz) ** 2
        d = jnp.minimum(d, dn)
        idxv = jnp.where(jota == i, nxt, idxv)
        return d, idxv

    d, idxv = jax.lax.fori_loop(1, n_s, body, (d, idxv))
    o_ref[...] = idxv


@partial(jax.jit, static_argnames=("n_s",))
def _fps(pos, n_s):
    N = pos.shape[0]
    Np = _pad_to(N, 128)
    p0 = pos[0]
    pad = jnp.broadcast_to(p0, (Np - N, 3))
    pp = jnp.concatenate([pos, pad], axis=0)
    pxyz = jnp.zeros((8, Np), jnp.float32).at[0:3, :].set(pp.T)
    pr = jnp.pad(pp, ((0, 0), (0, 5)))
    ns_pad = _pad_to(n_s, 128)
    out = pl.pallas_call(
        partial(_fps_kernel, n_s=n_s),
        out_shape=jax.ShapeDtypeStruct((1, ns_pad), jnp.int32),
    )(pxyz, pr)
    return out[0, :n_s]


def _mlp_masked(h, mask, layers):
    m = mask[..., None].astype(h.dtype)
    cnt = jnp.maximum(jnp.sum(m), 1.0)
    for L in layers:
        h = jax.nn.relu(h @ L["W"] + L["b"])
        mean = jnp.sum(h * m, axis=(0, 1)) / cnt
        var = jnp.sum(((h - mean) ** 2) * m, axis=(0, 1)) / cnt
        h = L["g"] * (h - mean) / jnp.sqrt(var + 1e-05) + L["be"]
    return h


def _mlp_dense(h, layers):
    for L in layers:
        h = jax.nn.relu(h @ L["W"] + L["b"])
        mean = jnp.mean(h, axis=0)
        var = jnp.mean((h - mean) ** 2, axis=0)
        h = L["g"] * (h - mean) / jnp.sqrt(var + 1e-05) + L["be"]
    return h


def _sa_module(x, pos, ratio, r, layers):
    N = pos.shape[0]
    n_s = int(math.ceil(ratio * N))
    idx = _fps(pos, n_s)
    pos_s = pos[idx]
    d2 = jnp.sum((pos_s[:, None, :] - pos[None, :, :]) ** 2, axis=-1)
    K = min(_MAXNB, N)
    score = jnp.where(d2 <= r * r, -d2, -jnp.inf)
    topv, topi = jax.lax.top_k(score, K)
    mask = topv > -jnp.inf
    rel = pos[topi] - pos_s[:, None, :]
    feat = jnp.concatenate([x[topi], rel], axis=-1)
    h = _mlp_masked(feat, mask, layers)
    h = jnp.where(mask[..., None], h, -jnp.inf)
    out = jnp.max(h, axis=1)
    out = jnp.where(jnp.isfinite(out), out, 0.0)
    return out, pos_s


def _knn_interpolate(x, pos_src, pos_dst, k):
    d2 = jnp.sum((pos_dst[:, None, :] - pos_src[None, :, :]) ** 2, axis=-1)
    kk = min(k, pos_src.shape[0])
    negd, nidx = jax.lax.top_k(-d2, kk)
    w = 1.0 / jnp.clip(-negd, 1e-16, None)
    num = jnp.sum(x[nidx] * w[..., None], axis=1)
    den = jnp.sum(w, axis=1, keepdims=True)
    return num / den


def _fp_module(x, pos, x_skip, pos_skip, k, layers):
    xi = _knn_interpolate(x, pos, pos_skip, k)
    xi = jnp.concatenate([xi, x_skip], axis=1)
    return _mlp_dense(xi, layers)


def kernel(x, pos, batch, params):
    xs = [x]
    ps = [pos]
    h, p = x, pos
    for i, (ratio, r, _) in enumerate(_SA_SPECS):
        h, p = _sa_module(h, p, ratio, r, params["sa%d" % (i + 1)])
        xs.append(h)
        ps.append(p)
    for j, (k, _) in enumerate(_FP_SPECS):
        lvl = 5 - j
        h = _fp_module(h, ps[lvl], xs[lvl - 1], ps[lvl - 1], k, params["fp%d" % lvl])
    h = jax.nn.relu(h @ params["lin1"]["W"] + params["lin1"]["b"])
    h = h @ params["lin2"]["W"] + params["lin2"]["b"]
    h = h @ params["lin3"]["W"] + params["lin3"]["b"]
    return h


# speedup vs baseline: 2.0682x; 1.0056x over previous
import math
from functools import partial

import jax
import jax.numpy as jnp
from jax.experimental import pallas as pl
from jax.experimental.pallas import tpu as pltpu

_INP_DIM = 3
_OUP_DIM = 13
_MAXNB = 128

_SA_SPECS = [
    (0.2, 0.05, [_INP_DIM + 3, 64, 64, 128]),
    (0.3, 0.1, [131, 128, 128, 128]),
    (0.3, 0.2, [131, 256, 256, 256]),
    (0.3, 0.4, [259, 512, 512, 512]),
    (0.3, 0.8, [515, 512, 512, 1024]),
]
_FP_SPECS = [
    (1, [1536, 512, 512]),
    (3, [768, 256, 256]),
    (3, [384, 256, 256]),
    (3, [384, 256, 128]),
    (3, [128 + _INP_DIM, 128, 128, 128]),
]


def _pad_to(n, m):
    return ((n + m - 1) // m) * m


def _fps_kernel(pxyz_ref, pr_ref, o_ref, *, n_s):
    X = pxyz_ref[0:1, :]
    Y = pxyz_ref[1:2, :]
    Z = pxyz_ref[2:3, :]
    Np = X.shape[1]
    iota = jax.lax.broadcasted_iota(jnp.int32, (1, Np), 1)
    r0 = pr_ref[0:1, :]
    px = r0[0:1, 0:1]
    py = r0[0:1, 1:2]
    pz = r0[0:1, 2:3]
    d = (X - px) ** 2 + (Y - py) ** 2 + (Z - pz) ** 2
    ns_pad = o_ref.shape[1]
    jota = jax.lax.broadcasted_iota(jnp.int32, (1, ns_pad), 1)
    idxv = jnp.zeros((1, ns_pad), jnp.int32)

    def body(i, carry):
        d, idxv = carry
        m = jnp.max(d)
        nxt = jnp.min(jnp.where(d == m, iota, jnp.int32(Np)))
        row = pr_ref[pl.ds(nxt, 1), :]
        qx = row[0:1, 0:1]
        qy = row[0:1, 1:2]
        qz = row[0:1, 2:3]
        dn = (X - qx) ** 2 + (Y - qy) ** 2 + (Z - qz) ** 2
        d = jnp.minimum(d, dn)
        idxv = jnp.where(jota == i, nxt, idxv)
        return d, idxv

    d, idxv = jax.lax.fori_loop(1, n_s, body, (d, idxv))
    o_ref[...] = idxv


@partial(jax.jit, static_argnames=("n_s",))
def _fps(pos, n_s):
    N = pos.shape[0]
    Np = _pad_to(N, 128)
    p0 = pos[0]
    pad = jnp.broadcast_to(p0, (Np - N, 3))
    pp = jnp.concatenate([pos, pad], axis=0)
    pxyz = jnp.zeros((8, Np), jnp.float32).at[0:3, :].set(pp.T)
    pr = jnp.pad(pp, ((0, 0), (0, 5)))
    ns_pad = _pad_to(n_s, 128)
    out = pl.pallas_call(
        partial(_fps_kernel, n_s=n_s),
        out_shape=jax.ShapeDtypeStruct((1, ns_pad), jnp.int32),
    )(pxyz, pr)
    return out[0, :n_s]


def _mlp_masked(h, mask, layers):
    m = mask[..., None].astype(h.dtype)
    cnt = jnp.maximum(jnp.sum(m), 1.0)
    for L in layers:
        h = jax.nn.relu(h @ L["W"] + L["b"])
        mean = jnp.sum(h * m, axis=(0, 1)) / cnt
        var = jnp.sum(((h - mean) ** 2) * m, axis=(0, 1)) / cnt
        h = L["g"] * (h - mean) / jnp.sqrt(var + 1e-05) + L["be"]
    return h


def _mlp_dense(h, layers):
    for L in layers:
        h = jax.nn.relu(h @ L["W"] + L["b"])
        mean = jnp.mean(h, axis=0)
        var = jnp.mean((h - mean) ** 2, axis=0)
        h = L["g"] * (h - mean) / jnp.sqrt(var + 1e-05) + L["be"]
    return h


def _sa_module(x, pos, ratio, r, layers):
    N = pos.shape[0]
    n_s = int(math.ceil(ratio * N))
    idx = _fps(pos, n_s)
    pos_s = pos[idx]
    d2 = jnp.sum((pos_s[:, None, :] - pos[None, :, :]) ** 2, axis=-1)
    K = min(_MAXNB, N)
    score = jnp.where(d2 <= r * r, -d2, -jnp.inf)
    topv, topi = jax.lax.approx_max_k(score, K, recall_target=1.0)
    mask = topv > -jnp.inf
    rel = pos[topi] - pos_s[:, None, :]
    feat = jnp.concatenate([x[topi], rel], axis=-1)
    h = _mlp_masked(feat, mask, layers)
    h = jnp.where(mask[..., None], h, -jnp.inf)
    out = jnp.max(h, axis=1)
    out = jnp.where(jnp.isfinite(out), out, 0.0)
    return out, pos_s


def _knn_interpolate(x, pos_src, pos_dst, k):
    d2 = jnp.sum((pos_dst[:, None, :] - pos_src[None, :, :]) ** 2, axis=-1)
    kk = min(k, pos_src.shape[0])
    negd, nidx = jax.lax.top_k(-d2, kk)
    w = 1.0 / jnp.clip(-negd, 1e-16, None)
    num = jnp.sum(x[nidx] * w[..., None], axis=1)
    den = jnp.sum(w, axis=1, keepdims=True)
    return num / den


def _fp_module(x, pos, x_skip, pos_skip, k, layers):
    xi = _knn_interpolate(x, pos, pos_skip, k)
    xi = jnp.concatenate([xi, x_skip], axis=1)
    return _mlp_dense(xi, layers)


def kernel(x, pos, batch, params):
    xs = [x]
    ps = [pos]
    h, p = x, pos
    for i, (ratio, r, _) in enumerate(_SA_SPECS):
        h, p = _sa_module(h, p, ratio, r, params["sa%d" % (i + 1)])
        xs.append(h)
        ps.append(p)
    for j, (k, _) in enumerate(_FP_SPECS):
        lvl = 5 - j
        h = _fp_module(h, ps[lvl], xs[lvl - 1], ps[lvl - 1], k, params["fp%d" % lvl])
    h = jax.nn.relu(h @ params["lin1"]["W"] + params["lin1"]["b"])
    h = h @ params["lin2"]["W"] + params["lin2"]["b"]
    h = h @ params["lin3"]["W"] + params["lin3"]["b"]
    return h


# speedup vs baseline: 6.1139x; 2.9561x over previous
import math
from functools import partial

import jax
import jax.numpy as jnp
from jax.experimental import pallas as pl
from jax.experimental.pallas import tpu as pltpu

_INP_DIM = 3
_OUP_DIM = 13
_MAXNB = 128

_SA_SPECS = [
    (0.2, 0.05, [_INP_DIM + 3, 64, 64, 128]),
    (0.3, 0.1, [131, 128, 128, 128]),
    (0.3, 0.2, [131, 256, 256, 256]),
    (0.3, 0.4, [259, 512, 512, 512]),
    (0.3, 0.8, [515, 512, 512, 1024]),
]
_FP_SPECS = [
    (1, [1536, 512, 512]),
    (3, [768, 256, 256]),
    (3, [384, 256, 256]),
    (3, [384, 256, 128]),
    (3, [128 + _INP_DIM, 128, 128, 128]),
]


def _pad_to(n, m):
    return ((n + m - 1) // m) * m


def _fps_kernel(pxyz_ref, pr_ref, o_ref, *, n_s):
    X = pxyz_ref[0:1, :]
    Y = pxyz_ref[1:2, :]
    Z = pxyz_ref[2:3, :]
    Np = X.shape[1]
    iota = jax.lax.broadcasted_iota(jnp.int32, (1, Np), 1)
    r0 = pr_ref[0:1, :]
    px = r0[0:1, 0:1]
    py = r0[0:1, 1:2]
    pz = r0[0:1, 2:3]
    d = (X - px) ** 2 + (Y - py) ** 2 + (Z - pz) ** 2
    ns_pad = o_ref.shape[1]
    jota = jax.lax.broadcasted_iota(jnp.int32, (1, ns_pad), 1)
    idxv = jnp.zeros((1, ns_pad), jnp.int32)

    def body(i, carry):
        d, idxv = carry
        m = jnp.max(d)
        nxt = jnp.min(jnp.where(d == m, iota, jnp.int32(Np)))
        row = pr_ref[pl.ds(nxt, 1), :]
        qx = row[0:1, 0:1]
        qy = row[0:1, 1:2]
        qz = row[0:1, 2:3]
        dn = (X - qx) ** 2 + (Y - qy) ** 2 + (Z - qz) ** 2
        d = jnp.minimum(d, dn)
        idxv = jnp.where(jota == i, nxt, idxv)
        return d, idxv

    d, idxv = jax.lax.fori_loop(1, n_s, body, (d, idxv))
    o_ref[...] = idxv


@partial(jax.jit, static_argnames=("n_s",))
def _fps(pos, n_s):
    N = pos.shape[0]
    Np = _pad_to(N, 128)
    p0 = pos[0]
    pad = jnp.broadcast_to(p0, (Np - N, 3))
    pp = jnp.concatenate([pos, pad], axis=0)
    pxyz = jnp.zeros((8, Np), jnp.float32).at[0:3, :].set(pp.T)
    pr = jnp.pad(pp, ((0, 0), (0, 5)))
    ns_pad = _pad_to(n_s, 128)
    out = pl.pallas_call(
        partial(_fps_kernel, n_s=n_s),
        out_shape=jax.ShapeDtypeStruct((1, ns_pad), jnp.int32),
    )(pxyz, pr)
    return out[0, :n_s]


def _nbr_kernel(pxyz_ref, qc_ref, idx_ref, msk_ref, *, K, r2, n_valid):
    _INF = jnp.float32(jnp.inf)
    X = pxyz_ref[0:1, :]
    Y = pxyz_ref[1:2, :]
    Z = pxyz_ref[2:3, :]
    Np = X.shape[1]
    q = qc_ref[...]
    qx = q[:, 0:1]
    qy = q[:, 1:2]
    qz = q[:, 2:3]
    d2 = (X - qx) ** 2 + (Y - qy) ** 2 + (Z - qz) ** 2
    iota = jax.lax.broadcasted_iota(jnp.int32, (8, Np), 1)
    D = jnp.where((d2 <= r2) & (iota < n_valid), d2, _INF)
    jota = jax.lax.broadcasted_iota(jnp.int32, (8, 128), 1)
    iv = jnp.zeros((8, 128), jnp.int32)
    mv = jnp.zeros((8, 128), jnp.int32)
    c = jnp.zeros((8, 1), jnp.int32)

    def cond(carry):
        return carry[4]

    def body(carry):
        D, iv, mv, c, _ = carry
        m = jnp.min(D, axis=1, keepdims=True)
        has = m < _INF
        hi = jnp.min(jnp.where(D == m, iota, jnp.int32(Np)), axis=1, keepdims=True)
        D = jnp.where(iota == hi, _INF, D)
        take = has & (c < K)
        upd = (jota == c) & take
        iv = jnp.where(upd, hi, iv)
        mv = jnp.where(upd, 1, mv)
        c = c + take.astype(jnp.int32)
        return (D, iv, mv, c, jnp.any(has))

    _, iv, mv, _, _ = jax.lax.while_loop(
        cond, body, (D, iv, mv, c, jnp.bool_(True)))
    idx_ref[...] = iv
    msk_ref[...] = mv


@partial(jax.jit, static_argnames=("K", "r2"))
def _radius_knn(pos, pos_s, K, r2):
    N = pos.shape[0]
    n_s = pos_s.shape[0]
    Np = _pad_to(N, 128)
    ns_pad = _pad_to(n_s, 8)
    pxyz = jnp.zeros((8, Np), jnp.float32).at[0:3, :N].set(pos.T)
    qc = jnp.full((ns_pad, 8), 1e6, jnp.float32).at[:n_s, 0:3].set(pos_s)
    idx, msk = pl.pallas_call(
        partial(_nbr_kernel, K=K, r2=r2, n_valid=N),
        grid=(ns_pad // 8,),
        in_specs=[
            pl.BlockSpec((8, Np), lambda i: (0, 0)),
            pl.BlockSpec((8, 8), lambda i: (i, 0)),
        ],
        out_specs=[
            pl.BlockSpec((8, 128), lambda i: (i, 0)),
            pl.BlockSpec((8, 128), lambda i: (i, 0)),
        ],
        out_shape=[
            jax.ShapeDtypeStruct((ns_pad, 128), jnp.int32),
            jax.ShapeDtypeStruct((ns_pad, 128), jnp.int32),
        ],
    )(pxyz, qc)
    return idx[:n_s, :K], msk[:n_s, :K] > 0


def _mlp_masked(h, mask, layers):
    m = mask[..., None].astype(h.dtype)
    cnt = jnp.maximum(jnp.sum(m), 1.0)
    for L in layers:
        h = jax.nn.relu(h @ L["W"] + L["b"])
        mean = jnp.sum(h * m, axis=(0, 1)) / cnt
        var = jnp.sum(((h - mean) ** 2) * m, axis=(0, 1)) / cnt
        h = L["g"] * (h - mean) / jnp.sqrt(var + 1e-05) + L["be"]
    return h


def _mlp_dense(h, layers):
    for L in layers:
        h = jax.nn.relu(h @ L["W"] + L["b"])
        mean = jnp.mean(h, axis=0)
        var = jnp.mean((h - mean) ** 2, axis=0)
        h = L["g"] * (h - mean) / jnp.sqrt(var + 1e-05) + L["be"]
    return h


def _sa_module(x, pos, ratio, r, layers):
    N = pos.shape[0]
    n_s = int(math.ceil(ratio * N))
    idx = _fps(pos, n_s)
    pos_s = pos[idx]
    K = min(_MAXNB, N)
    topi, mask = _radius_knn(pos, pos_s, K, r * r)
    rel = pos[topi] - pos_s[:, None, :]
    feat = jnp.concatenate([x[topi], rel], axis=-1)
    h = _mlp_masked(feat, mask, layers)
    h = jnp.where(mask[..., None], h, -jnp.inf)
    out = jnp.max(h, axis=1)
    out = jnp.where(jnp.isfinite(out), out, 0.0)
    return out, pos_s


def _knn_interpolate(x, pos_src, pos_dst, k):
    d2 = jnp.sum((pos_dst[:, None, :] - pos_src[None, :, :]) ** 2, axis=-1)
    kk = min(k, pos_src.shape[0])
    negd, nidx = jax.lax.top_k(-d2, kk)
    w = 1.0 / jnp.clip(-negd, 1e-16, None)
    num = jnp.sum(x[nidx] * w[..., None], axis=1)
    den = jnp.sum(w, axis=1, keepdims=True)
    return num / den


def _fp_module(x, pos, x_skip, pos_skip, k, layers):
    xi = _knn_interpolate(x, pos, pos_skip, k)
    xi = jnp.concatenate([xi, x_skip], axis=1)
    return _mlp_dense(xi, layers)


def kernel(x, pos, batch, params):
    xs = [x]
    ps = [pos]
    h, p = x, pos
    for i, (ratio, r, _) in enumerate(_SA_SPECS):
        h, p = _sa_module(h, p, ratio, r, params["sa%d" % (i + 1)])
        xs.append(h)
        ps.append(p)
    for j, (k, _) in enumerate(_FP_SPECS):
        lvl = 5 - j
        h = _fp_module(h, ps[lvl], xs[lvl - 1], ps[lvl - 1], k, params["fp%d" % lvl])
    h = jax.nn.relu(h @ params["lin1"]["W"] + params["lin1"]["b"])
    h = h @ params["lin2"]["W"] + params["lin2"]["b"]
    h = h @ params["lin3"]["W"] + params["lin3"]["b"]
    return h
